# baseline (speedup 1.0000x reference)
import functools

import jax
import jax.numpy as jnp
from jax.experimental import pallas as pl
from jax.experimental.pallas import tpu as pltpu

_S = 512


def _roll1(x):
    return pltpu.roll(x, 1, 1)


def _body(x_ref, u_ref, q_ref, ch_ref, loss_ref, *, R):
    i = pl.program_id(0)
    S = x_ref.shape[0]

    x = x_ref[...]
    u_c = u_ref[...]
    q = q_ref[...]

    lane = jax.lax.broadcasted_iota(jnp.int32, (S, 128), 1)
    m = lane % 4

    s1 = jnp.where(m == 1, x + _roll1(x), x)
    s2 = jnp.where(m == 2, x + _roll1(s1), s1)
    cdf = jnp.where(m == 3, x + _roll1(s2), s2)

    gi = jax.lax.broadcasted_iota(jnp.int32, (32, 128), 0)
    li = jax.lax.broadcasted_iota(jnp.int32, (32, 128), 1)
    E = (li // 4 == gi).astype(jnp.float32)
    li2 = jax.lax.broadcasted_iota(jnp.int32, (128, 32), 0)
    gi2 = jax.lax.broadcasted_iota(jnp.int32, (128, 32), 1)
    G = (li2 // 4 == gi2).astype(jnp.float32)

    uq = jax.lax.dot(u_c, E, preferred_element_type=jnp.float32, precision=jax.lax.Precision.HIGHEST)

    ind = jnp.where(uq > cdf, 1.0, 0.0)
    ch = jax.lax.dot(ind, G, preferred_element_type=jnp.float32, precision=jax.lax.Precision.HIGHEST)
    ch_ref[...] = ch.astype(jnp.int32)

    ge = 1.0 - ind
    geprev = jnp.where(m == 0, 0.0, _roll1(ge))
    first = ge * (1.0 - geprev)
    first = first + jnp.where(m == 3, ind, 0.0)

    rp = jax.lax.dot(x * first, G, preferred_element_type=jnp.float32, precision=jax.lax.Precision.HIGHEST)
    contrib = q * jnp.log(rp)

    srow = i * S + jax.lax.broadcasted_iota(jnp.int32, (S, 32), 0)
    contrib = jnp.where(srow < R, contrib, 0.0)
    partial = jnp.sum(contrib)

    @pl.when(i == 0)
    def _():
        loss_ref[0, 0] = 0.0

    loss_ref[0, 0] += partial


def kernel(pmfs, q_values):
    B, A = pmfs.shape
    assert A == 4 and B % 32 == 0
    u = jax.random.uniform(jax.random.key(1), (B,), dtype=jnp.float32)

    R = B // 32
    X = pmfs.reshape(R, 128)
    U2 = u.reshape(R, 32)
    Q2 = q_values.reshape(R, 32)
    grid = (pl.cdiv(R, _S),)

    ch2, lossacc = pl.pallas_call(
        functools.partial(_body, R=R),
        grid=grid,
        in_specs=[
            pl.BlockSpec((_S, 128), lambda i: (i, 0)),
            pl.BlockSpec((_S, 32), lambda i: (i, 0)),
            pl.BlockSpec((_S, 32), lambda i: (i, 0)),
        ],
        out_specs=[
            pl.BlockSpec((_S, 32), lambda i: (i, 0)),
            pl.BlockSpec(memory_space=pltpu.SMEM),
        ],
        out_shape=[
            jax.ShapeDtypeStruct((R, 32), jnp.int32),
            jax.ShapeDtypeStruct((1, 1), jnp.float32),
        ],
        compiler_params=pltpu.CompilerParams(
            dimension_semantics=("arbitrary",),
        ),
    )(X, U2, Q2)

    loss = -lossacc[0, 0] / B
    return (loss, ch2.reshape(B))


# speedup vs baseline: 19.6555x; 19.6555x over previous
import functools

import jax
import jax.numpy as jnp
from jax.experimental import pallas as pl
from jax.experimental.pallas import tpu as pltpu

_W = 16384


def _threefry2x32(c0, c1):
    ks0 = jnp.uint32(0)
    ks1 = jnp.uint32(1)
    ks2 = jnp.uint32(0x1BD11BDA) ^ ks0 ^ ks1

    x0 = c0 + ks0
    x1 = c1 + ks1
    rotations = ((13, 15, 26, 6), (17, 29, 16, 24))
    ks = (ks0, ks1, ks2)
    for i in range(5):
        for r in rotations[i % 2]:
            x0 = x0 + x1
            x1 = (x1 << r) | (x1 >> (32 - r))
            x1 = x1 ^ x0
        x0 = x0 + ks[(i + 1) % 3]
        x1 = x1 + ks[(i + 2) % 3] + jnp.uint32(i + 1)
    return x0, x1


def _bits_to_unit_float(bits):
    fb = (bits >> 9) | jnp.uint32(0x3F800000)
    return jax.lax.bitcast_convert_type(fb, jnp.float32) - 1.0


def _body(x1_ref, x2_ref, q1_ref, q2_ref, ch_ref, loss_ref, *, B, W):
    i = pl.program_id(0)
    H = B // 2
    a = i * (2 * W)

    x8 = jnp.concatenate([x1_ref[...], x2_ref[...]], axis=0)

    s_iota = jax.lax.broadcasted_iota(jnp.int32, (8, W), 0)
    g = s_iota % 4

    def roll1(v):
        return pltpu.roll(v, 1, 0)

    s1 = jnp.where(g == 1, x8 + roll1(x8), x8)
    s2 = jnp.where(g == 2, x8 + roll1(s1), s1)
    cdf = jnp.where(g == 3, x8 + roll1(s2), s2)

    Wc = W // 4
    ctr = jnp.uint32(a) + (
        jax.lax.broadcasted_iota(jnp.uint32, (8, Wc), 0) * jnp.uint32(Wc)
        + jax.lax.broadcasted_iota(jnp.uint32, (8, Wc), 1)
    )
    b0, b1 = _threefry2x32(jnp.zeros_like(ctr), ctr)
    u_c = _bits_to_unit_float(b0 ^ b1)

    u_flat = jnp.concatenate([u_c[s : s + 1, :] for s in range(8)], axis=1)
    u_lo = u_flat[:, :W]
    u_hi = u_flat[:, W:]
    ub = jnp.where(s_iota < 4, u_lo, u_hi)

    gt = ub > cdf
    ind = jnp.where(gt, 1.0, 0.0)

    def grpsum(v):
        b = v + jnp.where(g >= 1, roll1(v), 0.0)
        return b + jnp.where(g >= 2, pltpu.roll(b, 2, 0), 0.0)

    csum = grpsum(ind)
    ch_lo = csum[3, :].astype(jnp.int32)
    ch_hi = csum[7, :].astype(jnp.int32)
    ch_ref[pl.ds(0, W)] = ch_lo
    ch_ref[pl.ds(W, W)] = ch_hi

    ge = 1.0 - ind
    prev = jnp.where(g == 0, 0.0, roll1(ge))
    first = ge * (1.0 - prev) + jnp.where(g == 3, ind, 0.0)
    rpv = grpsum(x8 * first)

    rp2 = jnp.concatenate([rpv[3:4, :], rpv[7:8, :]], axis=0)
    qq = jnp.concatenate(
        [q1_ref[...].reshape(1, W), q2_ref[...].reshape(1, W)], axis=0
    )
    contrib = qq * jnp.log(rp2)

    l_iota = jax.lax.broadcasted_iota(jnp.int32, (2, W), 1)
    row = a + jax.lax.broadcasted_iota(jnp.int32, (2, W), 0) * W + l_iota
    contrib = jnp.where(row < B, contrib, 0.0)
    partial = jnp.sum(contrib)

    @pl.when(i == 0)
    def _():
        loss_ref[0, 0] = 0.0

    loss_ref[0, 0] += partial


def kernel(pmfs, q_values):
    B, A = pmfs.shape
    assert A == 4 and B % 2 == 0
    pt = pmfs.T
    W = _W
    grid = (pl.cdiv(B, 2 * W),)

    ch, lossacc = pl.pallas_call(
        functools.partial(_body, B=B, W=W),
        grid=grid,
        in_specs=[
            pl.BlockSpec((4, W), lambda i: (0, 2 * i)),
            pl.BlockSpec((4, W), lambda i: (0, 2 * i + 1)),
            pl.BlockSpec((W,), lambda i: (2 * i,)),
            pl.BlockSpec((W,), lambda i: (2 * i + 1,)),
        ],
        out_specs=[
            pl.BlockSpec((2 * W,), lambda i: (i,)),
            pl.BlockSpec(memory_space=pltpu.SMEM),
        ],
        out_shape=[
            jax.ShapeDtypeStruct((B,), jnp.int32),
            jax.ShapeDtypeStruct((1, 1), jnp.float32),
        ],
        compiler_params=pltpu.CompilerParams(
            dimension_semantics=("arbitrary",),
        ),
    )(pt, pt, q_values, q_values)

    loss = -lossacc[0, 0] / B
    return (loss, ch)
